# baseline (speedup 1.0000x reference)
import jax
import jax.numpy as jnp
from jax import lax
from jax.experimental import pallas as pl
from jax.experimental.pallas import tpu as pltpu
from jax.experimental.pallas import tpu_sc as plsc

_B = 1638400
_V = 1000000
_NC = 2
_NS = 16
_NW = _NC * _NS
_BPW = _B // _NW
_L = 16

_CH = 62496
_CH_LAST = _V - 15 * _CH
_SCH = 5208
_SNJ = _CH // _SCH
_SCH_LAST = 6256
_SNJ_LAST = _CH_LAST // _SCH_LAST

_GC = 6400
_NG = _BPW // _GC

_NH = 5


def _landmark_body(counts_hbm, obs_hbm, idx_hbm, out_hbm, idx_v, buf_v,
                   obs_v, table_sh, ssem, isem, gsem0, gsem1, osem0, osem1):
    c = lax.axis_index("c")
    s = lax.axis_index("s")
    wid = s * _NC + c
    base = wid * _BPW

    icp = pltpu.async_copy(idx_hbm.at[pl.ds(base, _BPW)], idx_v, isem)
    off = s * _CH

    def stage(sch, snj):
        si = [None, None]
        so = [None, None]
        for j in range(snj):
            b = j % 2
            if j >= 2:
                so[b].wait()
            si[b] = pltpu.async_copy(
                counts_hbm.at[pl.ds(off + j * sch, sch)],
                buf_v.at[pl.ds(b * _GC, sch)], gsem0 if b == 0 else gsem1)
            if j >= 1:
                pb = 1 - b
                si[pb].wait()
                so[pb] = pltpu.async_copy(
                    buf_v.at[pl.ds(pb * _GC, sch)],
                    table_sh.at[pl.ds(off + (j - 1) * sch, sch)],
                    osem0 if pb == 0 else osem1)
        lb = (snj - 1) % 2
        si[lb].wait()
        if snj >= 2:
            so[1 - lb].wait()
        pltpu.async_copy(
            buf_v.at[pl.ds(lb * _GC, sch)],
            table_sh.at[pl.ds(off + (snj - 1) * sch, sch)], ssem).wait()

    with jax.named_scope("stage_table"):
        @pl.when(s < _NS - 1)
        def _():
            stage(_SCH, _SNJ)

        @pl.when(s == _NS - 1)
        def _():
            stage(_SCH_LAST, _SNJ_LAST)

        pltpu.sync_copy(obs_hbm, obs_v.at[pl.ds(0, 1)])
        recip = (1.0 / obs_v[pl.ds(0, _L)])[0]
        icp.wait()
    with jax.named_scope("stage_barrier"):
        plsc.subcore_barrier()

    gsem = (gsem0, gsem1)
    osem = (osem0, osem1)

    def scale_buf(b):
        def body(i, carry):
            sl = pl.ds(b * _GC + i * _L, _L)
            buf_v[sl] = buf_v[sl] * recip
            return carry
        lax.fori_loop(0, _GC // _L, body, 0)

    def gather_pipeline(src):
        g = [None, None]
        o = [None, None]
        for j in range(_NG):
            b = j % 2
            if j >= 2:
                o[b].wait()
            g[b] = pltpu.async_copy(
                src.at[idx_v.at[pl.ds(j * _GC, _GC)]],
                buf_v.at[pl.ds(b * _GC, _GC)], gsem[b])
            if j >= 1:
                pb = 1 - b
                g[pb].wait()
                scale_buf(pb)
                o[pb] = pltpu.async_copy(
                    buf_v.at[pl.ds(pb * _GC, _GC)],
                    out_hbm.at[pl.ds(base + (j - 1) * _GC, _GC)], osem[pb])
        lb = (_NG - 1) % 2
        g[lb].wait()
        scale_buf(lb)
        o[1 - lb].wait()
        pltpu.async_copy(
            buf_v.at[pl.ds(lb * _GC, _GC)],
            out_hbm.at[pl.ds(base + (_NG - 1) * _GC, _GC)], osem[lb]).wait()

    with jax.named_scope("gather_loop"):
        @pl.when(s < _NH)
        def _():
            gather_pipeline(counts_hbm)

        @pl.when(s >= _NH)
        def _():
            gather_pipeline(table_sh)


def kernel(counts, obs_count, landmark_indices):
    mesh = plsc.VectorSubcoreMesh(core_axis_name="c", subcore_axis_name="s")
    k = pl.kernel(
        _landmark_body,
        mesh=mesh,
        out_type=jax.ShapeDtypeStruct((_B,), jnp.float32),
        scratch_types=[
            pltpu.VMEM((_BPW,), jnp.int32),
            pltpu.VMEM((2 * _GC,), jnp.float32),
            pltpu.VMEM((_L,), jnp.float32),
            pltpu.VMEM_SHARED((_V,), jnp.float32),
            pltpu.SemaphoreType.DMA,
            pltpu.SemaphoreType.DMA,
            pltpu.SemaphoreType.DMA,
            pltpu.SemaphoreType.DMA,
            pltpu.SemaphoreType.DMA,
            pltpu.SemaphoreType.DMA,
        ],
    )
    return k(counts, obs_count, landmark_indices)


# speedup vs baseline: 1.5845x; 1.5845x over previous
import jax
import jax.numpy as jnp
from jax import lax
from jax.experimental import pallas as pl
from jax.experimental.pallas import tpu as pltpu
from jax.experimental.pallas import tpu_sc as plsc

_B = 1638400
_V = 1000000
_NC = 2
_NS = 16
_NW = _NC * _NS
_BPW = _B // _NW
_L = 16

_CH = 62496
_CH_LAST = _V - 15 * _CH
_SCH = 3472
_SNJ = _CH // _SCH
_SCH_LAST = 3680
_SNJ_LAST = _CH_LAST // _SCH_LAST

_NG = 8
_GS = 4400
_GH = 2000
_SPLIT = _NG * _GS


def _landmark_body(counts_hbm, obs_hbm, idx_hbm, out_hbm, idx_v, buf_v,
                   obs_v, table_sh, isem, ssem, gs0, gs1, gh0, gh1,
                   os0, os1, oh0, oh1):
    c = lax.axis_index("c")
    s = lax.axis_index("s")
    wid = s * _NC + c
    base = wid * _BPW

    def sbuf(b):
        return buf_v.at[pl.ds(b * _GS, _GS)]

    def hbuf(b):
        return buf_v.at[pl.ds(2 * _GS + b * _GH, _GH)]

    icp = pltpu.async_copy(idx_hbm.at[pl.ds(base, _BPW)], idx_v, isem)
    off = s * _CH
    gsem = (gs0, gs1)
    osem = (os0, os1)

    def stage(sch, snj):
        si = [None, None]
        so = [None, None]
        for j in range(snj):
            b = j % 2
            if j >= 2:
                so[b].wait()
            si[b] = pltpu.async_copy(
                counts_hbm.at[pl.ds(off + j * sch, sch)],
                buf_v.at[pl.ds(b * _GS, sch)], gsem[b])
            if j >= 1:
                pb = 1 - b
                si[pb].wait()
                so[pb] = pltpu.async_copy(
                    buf_v.at[pl.ds(pb * _GS, sch)],
                    table_sh.at[pl.ds(off + (j - 1) * sch, sch)], osem[pb])
        lb = (snj - 1) % 2
        si[lb].wait()
        so[1 - lb].wait()
        pltpu.async_copy(
            buf_v.at[pl.ds(lb * _GS, sch)],
            table_sh.at[pl.ds(off + (snj - 1) * sch, sch)], ssem).wait()

    with jax.named_scope("stage_table"):
        @pl.when(s < _NS - 1)
        def _():
            stage(_SCH, _SNJ)

        @pl.when(s == _NS - 1)
        def _():
            stage(_SCH_LAST, _SNJ_LAST)

        pltpu.sync_copy(obs_hbm, obs_v.at[pl.ds(0, 1)])
        recip = (1.0 / obs_v[pl.ds(0, _L)])[0]
        icp.wait()
    with jax.named_scope("stage_barrier"):
        plsc.subcore_barrier()

    def scale_buf(word_off, n):
        def body(i, carry):
            sl = pl.ds(word_off + i * _L, _L)
            buf_v[sl] = buf_v[sl] * recip
            return carry
        lax.fori_loop(0, n // _L, body, 0)

    gS = [None, None]
    oS = [None, None]
    gH = [None, None]
    oH = [None, None]
    hsem = (gh0, gh1)
    hosem = (oh0, oh1)

    with jax.named_scope("gather_loop"):
        for j in range(_NG):
            b = j % 2
            pb = 1 - b
            if j >= 2:
                oS[b].wait()
                oH[b].wait()
            gS[b] = pltpu.async_copy(
                table_sh.at[idx_v.at[pl.ds(j * _GS, _GS)]], sbuf(b), gsem[b])
            gH[b] = pltpu.async_copy(
                counts_hbm.at[idx_v.at[pl.ds(_SPLIT + j * _GH, _GH)]],
                hbuf(b), hsem[b])
            if j >= 1:
                gS[pb].wait()
                scale_buf(pb * _GS, _GS)
                oS[pb] = pltpu.async_copy(
                    sbuf(pb),
                    out_hbm.at[pl.ds(base + (j - 1) * _GS, _GS)], osem[pb])
                gH[pb].wait()
                scale_buf(2 * _GS + pb * _GH, _GH)
                oH[pb] = pltpu.async_copy(
                    hbuf(pb),
                    out_hbm.at[pl.ds(base + _SPLIT + (j - 1) * _GH, _GH)],
                    hosem[pb])

    with jax.named_scope("gather_drain"):
        lb = (_NG - 1) % 2
        gS[lb].wait()
        scale_buf(lb * _GS, _GS)
        oS[1 - lb].wait()
        ol_s = pltpu.async_copy(
            sbuf(lb), out_hbm.at[pl.ds(base + (_NG - 1) * _GS, _GS)],
            osem[lb])
        gH[lb].wait()
        scale_buf(2 * _GS + lb * _GH, _GH)
        oH[1 - lb].wait()
        pltpu.async_copy(
            hbuf(lb),
            out_hbm.at[pl.ds(base + _SPLIT + (_NG - 1) * _GH, _GH)],
            hosem[lb]).wait()
        ol_s.wait()


def kernel(counts, obs_count, landmark_indices):
    mesh = plsc.VectorSubcoreMesh(core_axis_name="c", subcore_axis_name="s")
    k = pl.kernel(
        _landmark_body,
        mesh=mesh,
        out_type=jax.ShapeDtypeStruct((_B,), jnp.float32),
        scratch_types=[
            pltpu.VMEM((_BPW,), jnp.int32),
            pltpu.VMEM((2 * _GS + 2 * _GH,), jnp.float32),
            pltpu.VMEM((_L,), jnp.float32),
            pltpu.VMEM_SHARED((_V,), jnp.float32),
            pltpu.SemaphoreType.DMA,
            pltpu.SemaphoreType.DMA,
            pltpu.SemaphoreType.DMA,
            pltpu.SemaphoreType.DMA,
            pltpu.SemaphoreType.DMA,
            pltpu.SemaphoreType.DMA,
            pltpu.SemaphoreType.DMA,
            pltpu.SemaphoreType.DMA,
            pltpu.SemaphoreType.DMA,
            pltpu.SemaphoreType.DMA,
        ],
    )
    return k(counts, obs_count, landmark_indices)


# speedup vs baseline: 1.6503x; 1.0415x over previous
import jax
import jax.numpy as jnp
from jax import lax
from jax.experimental import pallas as pl
from jax.experimental.pallas import tpu as pltpu
from jax.experimental.pallas import tpu_sc as plsc

_B = 1638400
_V = 1000000
_NC = 2
_NS = 16
_NW = _NC * _NS
_BPW = _B // _NW
_L = 16

_CH = 62496
_CH_LAST = _V - 15 * _CH
_SCH = 5208
_SNJ = _CH // _SCH
_SCH_LAST = 6256
_SNJ_LAST = _CH_LAST // _SCH_LAST
_SHALF = 6400

_NG = 8
_GS = 4400
_GH = 2000
_SPLIT = _NG * _GS


def _landmark_body(counts_hbm, obs_hbm, idx_hbm, out_hbm, idx_v, buf_v,
                   obs_v, table_sh, isem, ssem, gs0, gs1, gh0, gh1,
                   os0, os1, oh0, oh1):
    c = lax.axis_index("c")
    s = lax.axis_index("s")
    wid = s * _NC + c
    base = wid * _BPW

    def sbuf(b):
        return buf_v.at[pl.ds(b * _GS, _GS)]

    def hbuf(b):
        return buf_v.at[pl.ds(2 * _GS + b * _GH, _GH)]

    icp = pltpu.async_copy(idx_hbm.at[pl.ds(base, _BPW)], idx_v, isem)
    off = s * _CH
    gsem = (gs0, gs1)
    osem = (os0, os1)

    def stage(sch, snj):
        si = [None, None]
        so = [None, None]
        for j in range(snj):
            b = j % 2
            if j >= 2:
                so[b].wait()
            si[b] = pltpu.async_copy(
                counts_hbm.at[pl.ds(off + j * sch, sch)],
                buf_v.at[pl.ds(b * _SHALF, sch)], gsem[b])
            if j >= 1:
                pb = 1 - b
                si[pb].wait()
                so[pb] = pltpu.async_copy(
                    buf_v.at[pl.ds(pb * _SHALF, sch)],
                    table_sh.at[pl.ds(off + (j - 1) * sch, sch)], osem[pb])
        lb = (snj - 1) % 2
        si[lb].wait()
        so[1 - lb].wait()
        pltpu.async_copy(
            buf_v.at[pl.ds(lb * _SHALF, sch)],
            table_sh.at[pl.ds(off + (snj - 1) * sch, sch)], ssem).wait()

    with jax.named_scope("stage_table"):
        @pl.when(s < _NS - 1)
        def _():
            stage(_SCH, _SNJ)

        @pl.when(s == _NS - 1)
        def _():
            stage(_SCH_LAST, _SNJ_LAST)

        pltpu.sync_copy(obs_hbm, obs_v.at[pl.ds(0, 1)])
        recip = (1.0 / obs_v[pl.ds(0, _L)])[0]
        icp.wait()
    with jax.named_scope("stage_barrier"):
        plsc.subcore_barrier()

    def scale_buf(word_off, n):
        def body(i, carry):
            sl = pl.ds(word_off + i * _L, _L)
            buf_v[sl] = buf_v[sl] * recip
            return carry
        lax.fori_loop(0, n // _L, body, 0)

    gS = [None, None]
    oS = [None, None]
    gH = [None, None]
    oH = [None, None]
    hsem = (gh0, gh1)
    hosem = (oh0, oh1)

    with jax.named_scope("gather_loop"):
        for j in range(_NG):
            b = j % 2
            pb = 1 - b
            if j >= 2:
                oS[b].wait()
                oH[b].wait()
            gS[b] = pltpu.async_copy(
                table_sh.at[idx_v.at[pl.ds(j * _GS, _GS)]], sbuf(b), gsem[b])
            gH[b] = pltpu.async_copy(
                counts_hbm.at[idx_v.at[pl.ds(_SPLIT + j * _GH, _GH)]],
                hbuf(b), hsem[b])
            if j >= 1:
                gS[pb].wait()
                scale_buf(pb * _GS, _GS)
                oS[pb] = pltpu.async_copy(
                    sbuf(pb),
                    out_hbm.at[pl.ds(base + (j - 1) * _GS, _GS)], osem[pb])
                gH[pb].wait()
                scale_buf(2 * _GS + pb * _GH, _GH)
                oH[pb] = pltpu.async_copy(
                    hbuf(pb),
                    out_hbm.at[pl.ds(base + _SPLIT + (j - 1) * _GH, _GH)],
                    hosem[pb])

    with jax.named_scope("gather_drain"):
        lb = (_NG - 1) % 2
        gS[lb].wait()
        scale_buf(lb * _GS, _GS)
        oS[1 - lb].wait()
        ol_s = pltpu.async_copy(
            sbuf(lb), out_hbm.at[pl.ds(base + (_NG - 1) * _GS, _GS)],
            osem[lb])
        gH[lb].wait()
        scale_buf(2 * _GS + lb * _GH, _GH)
        oH[1 - lb].wait()
        pltpu.async_copy(
            hbuf(lb),
            out_hbm.at[pl.ds(base + _SPLIT + (_NG - 1) * _GH, _GH)],
            hosem[lb]).wait()
        ol_s.wait()


def kernel(counts, obs_count, landmark_indices):
    mesh = plsc.VectorSubcoreMesh(core_axis_name="c", subcore_axis_name="s")
    k = pl.kernel(
        _landmark_body,
        mesh=mesh,
        out_type=jax.ShapeDtypeStruct((_B,), jnp.float32),
        scratch_types=[
            pltpu.VMEM((_BPW,), jnp.int32),
            pltpu.VMEM((2 * _GS + 2 * _GH,), jnp.float32),
            pltpu.VMEM((_L,), jnp.float32),
            pltpu.VMEM_SHARED((_V,), jnp.float32),
            pltpu.SemaphoreType.DMA,
            pltpu.SemaphoreType.DMA,
            pltpu.SemaphoreType.DMA,
            pltpu.SemaphoreType.DMA,
            pltpu.SemaphoreType.DMA,
            pltpu.SemaphoreType.DMA,
            pltpu.SemaphoreType.DMA,
            pltpu.SemaphoreType.DMA,
            pltpu.SemaphoreType.DMA,
            pltpu.SemaphoreType.DMA,
        ],
    )
    return k(counts, obs_count, landmark_indices)
